# baseline (speedup 1.0000x reference)
import jax
import jax.numpy as jnp
from jax.experimental import pallas as pl
from jax.experimental.pallas import tpu as pltpu

_TOP_K = 2
_ROUTED_SCALING = 2.5
_TILE = 256


def _moe_tile_kernel(te_ref, x_ref, wg_ref, wu_ref, wd_ref, o_ref):
    x = x_ref[...]
    g = jnp.dot(x, wg_ref[0], preferred_element_type=jnp.float32)
    u = jnp.dot(x, wu_ref[0], preferred_element_type=jnp.float32)
    h = (g * jax.nn.sigmoid(g)) * u
    o_ref[...] = jnp.dot(h.astype(jnp.bfloat16), wd_ref[0],
                         preferred_element_type=jnp.float32)


def kernel(hidden_states, gate_weight, gate_bias, all_gate_proj,
           all_up_proj, all_down_proj, shared_gate, shared_up, shared_down):
    orig_shape = hidden_states.shape
    D = orig_shape[-1]
    h = hidden_states.reshape(-1, D)
    T = h.shape[0]
    E, _, F = all_gate_proj.shape
    FS = shared_gate.shape[1]
    NSH = FS // F
    EA = E + NSH
    P = T * _TOP_K

    scores = jax.nn.sigmoid(h @ gate_weight)
    _, topk_idx = jax.lax.top_k(scores + gate_bias[None, :], _TOP_K)
    topk_w = jnp.take_along_axis(scores, topk_idx, axis=1)
    topk_w = topk_w / (jnp.sum(topk_w, axis=-1, keepdims=True) + 1e-20)
    topk_w = topk_w * _ROUTED_SCALING

    e_flat = topk_idx.reshape(-1).astype(jnp.int32)
    onehot = jax.nn.one_hot(e_flat, E, dtype=jnp.int32)
    cum = jnp.cumsum(onehot, axis=0)
    rank = jnp.take_along_axis(cum, e_flat[:, None], axis=1)[:, 0] - 1
    gsz = cum[-1]

    n_routed_tiles = P // _TILE + E
    n_shared_tiles = NSH * (T // _TILE)
    max_tiles = n_routed_tiles + n_shared_tiles
    npad = max_tiles * _TILE

    nt = jnp.concatenate([
        (gsz + _TILE - 1) // _TILE,
        jnp.full((NSH,), T // _TILE, dtype=jnp.int32),
    ]).astype(jnp.int32)
    nt_cum = jnp.cumsum(nt)
    pad_base = (nt_cum - nt) * _TILE
    tile_expert = jnp.minimum(
        jnp.searchsorted(nt_cum, jnp.arange(max_tiles, dtype=jnp.int32),
                         side='right'),
        EA - 1).astype(jnp.int32)

    dest_routed = pad_base[e_flat] + rank
    t_ar = jnp.arange(T, dtype=jnp.int32)
    dest_shared = (pad_base[E + jnp.arange(NSH)][None, :]
                   + t_ar[:, None])

    slot_token = jnp.zeros((npad,), jnp.int32)
    tok_of_pair = jnp.arange(P, dtype=jnp.int32) // _TOP_K
    slot_token = slot_token.at[dest_routed].set(tok_of_pair)
    slot_token = slot_token.at[dest_shared.reshape(-1)].set(
        jnp.repeat(t_ar, NSH))

    x_pad = h.astype(jnp.bfloat16)[slot_token]
    wg_aug = jnp.concatenate(
        [all_gate_proj, shared_gate.reshape(D, NSH, F).transpose(1, 0, 2)],
        axis=0).astype(jnp.bfloat16)
    wu_aug = jnp.concatenate(
        [all_up_proj, shared_up.reshape(D, NSH, F).transpose(1, 0, 2)],
        axis=0).astype(jnp.bfloat16)
    wd_aug = jnp.concatenate(
        [all_down_proj, shared_down.reshape(NSH, F, D)],
        axis=0).astype(jnp.bfloat16)

    grid_spec = pltpu.PrefetchScalarGridSpec(
        num_scalar_prefetch=1,
        grid=(max_tiles,),
        in_specs=[
            pl.BlockSpec((_TILE, D), lambda i, te: (i, 0)),
            pl.BlockSpec((1, D, F), lambda i, te: (te[i], 0, 0)),
            pl.BlockSpec((1, D, F), lambda i, te: (te[i], 0, 0)),
            pl.BlockSpec((1, F, D), lambda i, te: (te[i], 0, 0)),
        ],
        out_specs=pl.BlockSpec((_TILE, D), lambda i, te: (i, 0)),
    )
    y_pad = pl.pallas_call(
        _moe_tile_kernel,
        grid_spec=grid_spec,
        out_shape=jax.ShapeDtypeStruct((npad, D), jnp.float32),
    )(tile_expert, x_pad, wg_aug, wu_aug, wd_aug)

    y_routed = y_pad[dest_routed.reshape(T, _TOP_K)]
    out = jnp.sum(topk_w[:, :, None] * y_routed, axis=1)
    for j in range(NSH):
        out = out + y_pad[dest_shared[:, j]]
    return out.reshape(orig_shape)


# speedup vs baseline: 1.2946x; 1.2946x over previous
import jax
import jax.numpy as jnp
from jax.experimental import pallas as pl
from jax.experimental.pallas import tpu as pltpu

_TOP_K = 2
_ROUTED_SCALING = 2.5
_TILE = 256


def _moe_tile_kernel(te_ref, x_ref, wg_ref, wu_ref, wd_ref, o_ref):
    x = x_ref[...]
    g = jnp.dot(x, wg_ref[0], preferred_element_type=jnp.float32)
    u = jnp.dot(x, wu_ref[0], preferred_element_type=jnp.float32)
    h = (g * jax.nn.sigmoid(g)) * u
    o_ref[...] = jnp.dot(h.astype(jnp.bfloat16), wd_ref[0],
                         preferred_element_type=jnp.float32).astype(jnp.bfloat16)


def _shared_tile_kernel(x_ref, sg_ref, su_ref, sd_ref, o_ref):
    x = x_ref[...]
    g = jnp.dot(x, sg_ref[...], preferred_element_type=jnp.float32)
    u = jnp.dot(x, su_ref[...], preferred_element_type=jnp.float32)
    h = (g * jax.nn.sigmoid(g)) * u
    o_ref[...] = jnp.dot(h.astype(jnp.bfloat16), sd_ref[...],
                         preferred_element_type=jnp.float32)


def kernel(hidden_states, gate_weight, gate_bias, all_gate_proj,
           all_up_proj, all_down_proj, shared_gate, shared_up, shared_down):
    orig_shape = hidden_states.shape
    D = orig_shape[-1]
    h = hidden_states.reshape(-1, D)
    T = h.shape[0]
    E, _, F = all_gate_proj.shape
    FS = shared_gate.shape[1]
    P = T * _TOP_K

    scores = jax.nn.sigmoid(h @ gate_weight)
    _, topk_idx = jax.lax.top_k(scores + gate_bias[None, :], _TOP_K)
    topk_w = jnp.take_along_axis(scores, topk_idx, axis=1)
    topk_w = topk_w / (jnp.sum(topk_w, axis=-1, keepdims=True) + 1e-20)
    topk_w = topk_w * _ROUTED_SCALING

    e_flat = topk_idx.reshape(-1).astype(jnp.int32)
    onehot_t = (e_flat[None, :] == jnp.arange(E, dtype=jnp.int32)[:, None]
                ).astype(jnp.int32)
    cum_t = jnp.cumsum(onehot_t, axis=1)
    rank = jnp.take_along_axis(cum_t, e_flat[None, :], axis=0)[0] - 1
    gsz = cum_t[:, -1]

    max_tiles = P // _TILE + E
    npad = max_tiles * _TILE

    nt = (gsz + _TILE - 1) // _TILE
    nt_cum = jnp.cumsum(nt)
    pad_base = (nt_cum - nt) * _TILE
    tile_expert = jnp.minimum(
        jnp.searchsorted(nt_cum, jnp.arange(max_tiles, dtype=jnp.int32),
                         side='right'),
        E - 1).astype(jnp.int32)

    dest = pad_base[e_flat] + rank
    slot_token = jnp.zeros((npad,), jnp.int32)
    tok_of_pair = jnp.arange(P, dtype=jnp.int32) // _TOP_K
    slot_token = slot_token.at[dest].set(tok_of_pair)

    h_b16 = h.astype(jnp.bfloat16)
    x_pad = h_b16[slot_token]
    wg = all_gate_proj.astype(jnp.bfloat16)
    wu = all_up_proj.astype(jnp.bfloat16)
    wd = all_down_proj.astype(jnp.bfloat16)

    grid_spec = pltpu.PrefetchScalarGridSpec(
        num_scalar_prefetch=1,
        grid=(max_tiles,),
        in_specs=[
            pl.BlockSpec((_TILE, D), lambda i, te: (i, 0)),
            pl.BlockSpec((1, D, F), lambda i, te: (te[i], 0, 0)),
            pl.BlockSpec((1, D, F), lambda i, te: (te[i], 0, 0)),
            pl.BlockSpec((1, F, D), lambda i, te: (te[i], 0, 0)),
        ],
        out_specs=pl.BlockSpec((_TILE, D), lambda i, te: (i, 0)),
    )
    y_pad = pl.pallas_call(
        _moe_tile_kernel,
        grid_spec=grid_spec,
        out_shape=jax.ShapeDtypeStruct((npad, D), jnp.bfloat16),
    )(tile_expert, x_pad, wg, wu, wd)

    shared_out = pl.pallas_call(
        _shared_tile_kernel,
        grid=(T // _TILE,),
        in_specs=[
            pl.BlockSpec((_TILE, D), lambda i: (i, 0)),
            pl.BlockSpec((D, FS), lambda i: (0, 0)),
            pl.BlockSpec((D, FS), lambda i: (0, 0)),
            pl.BlockSpec((FS, D), lambda i: (0, 0)),
        ],
        out_specs=pl.BlockSpec((_TILE, D), lambda i: (i, 0)),
        out_shape=jax.ShapeDtypeStruct((T, D), jnp.float32),
    )(h_b16, shared_gate.astype(jnp.bfloat16),
      shared_up.astype(jnp.bfloat16), shared_down.astype(jnp.bfloat16))

    y_routed = y_pad[dest.reshape(T, _TOP_K)].astype(jnp.float32)
    out = shared_out + jnp.sum(topk_w[:, :, None] * y_routed, axis=1)
    return out.reshape(orig_shape)
